# baseline (speedup 1.0000x reference)
import jax
import jax.numpy as jnp
from jax.experimental import pallas as pl
from jax.experimental.pallas import tpu as pltpu

EPS_ = 1e-8


def _cln_kernel(x_ref, g_ref, b_ref, o_ref, carry_ref):
    j = pl.program_id(1)
    _, M, T = x_ref.shape

    x = x_ref[0]
    s = jnp.sum(x, axis=0, keepdims=True)
    s2 = jnp.sum(x * x, axis=0, keepdims=True)
    v = jnp.concatenate([s, s2], axis=0)

    k = 1
    while k < T:
        shifted = jnp.concatenate(
            [jnp.zeros((2, k), jnp.float32), v[:, : T - k]], axis=1)
        v = v + shifted
        k *= 2

    @pl.when(j == 0)
    def _():
        carry_ref[...] = jnp.zeros_like(carry_ref)

    v = v + carry_ref[...]
    carry_ref[...] = v[:, T - 1:T]

    pos0 = (j * T + 1).astype(jnp.float32)
    iota = jax.lax.broadcasted_iota(jnp.int32, (1, T), 1).astype(jnp.float32)
    cnt = (iota + pos0) * jnp.float32(M)
    inv = 1.0 / cnt
    mean = v[0:1] * inv
    ex2 = v[1:2] * inv
    var = ex2 - mean * mean
    rstd = jax.lax.rsqrt(var + EPS_)

    g = g_ref[0]
    b = b_ref[0]
    o_ref[0] = (x - mean) * (g * rstd) + b


def kernel(x, gamma, beta):
    B, M, L = x.shape
    T = 640
    nj = L // T
    grid = (B, nj)

    return pl.pallas_call(
        _cln_kernel,
        out_shape=jax.ShapeDtypeStruct((B, M, L), x.dtype),
        grid=grid,
        in_specs=[
            pl.BlockSpec((1, M, T), lambda b, j: (b, 0, j)),
            pl.BlockSpec((1, M, 1), lambda b, j: (b, 0, 0)),
            pl.BlockSpec((1, M, 1), lambda b, j: (b, 0, 0)),
        ],
        out_specs=pl.BlockSpec((1, M, T), lambda b, j: (b, 0, j)),
        scratch_shapes=[pltpu.VMEM((2, 1), jnp.float32)],
        compiler_params=pltpu.CompilerParams(
            dimension_semantics=("parallel", "arbitrary"),
        ),
        name="cum_layer_norm",
    )(x, gamma, beta)


# speedup vs baseline: 1.0210x; 1.0210x over previous
import jax
import jax.numpy as jnp
from jax.experimental import pallas as pl
from jax.experimental.pallas import tpu as pltpu

EPS_ = 1e-8


def _cln_kernel(x_ref, g_ref, b_ref, o_ref, carry_ref):
    j = pl.program_id(1)
    _, M, T = x_ref.shape

    R1 = 64
    acc = jnp.zeros((8, T), jnp.float32)
    acc2 = jnp.zeros((8, T), jnp.float32)
    for r in range(0, M, R1):
        xs = x_ref[0, r:r + R1, :].reshape(R1 // 8, 8, T)
        acc = acc + jnp.sum(xs, axis=0)
        acc2 = acc2 + jnp.sum(xs * xs, axis=0)
    s = jnp.sum(acc, axis=0, keepdims=True)
    s2 = jnp.sum(acc2, axis=0, keepdims=True)
    v = jnp.concatenate([s, s2], axis=0)

    k = 1
    while k < T:
        shifted = jnp.concatenate(
            [jnp.zeros((2, k), jnp.float32), v[:, : T - k]], axis=1)
        v = v + shifted
        k *= 2

    @pl.when(j == 0)
    def _():
        carry_ref[...] = jnp.zeros_like(carry_ref)

    v = v + carry_ref[...]
    carry_ref[...] = v[:, T - 1:T]

    pos0 = (j * T + 1).astype(jnp.float32)
    iota = jax.lax.broadcasted_iota(jnp.int32, (1, T), 1).astype(jnp.float32)
    cnt = (iota + pos0) * jnp.float32(M)
    inv = 1.0 / cnt
    mean = v[0:1] * inv
    ex2 = v[1:2] * inv
    var = ex2 - mean * mean
    rstd = jax.lax.rsqrt(var + EPS_)

    R2 = 64
    for r in range(0, M, R2):
        xs = x_ref[0, r:r + R2, :]
        gs = g_ref[0, r:r + R2, :]
        bs = b_ref[0, r:r + R2, :]
        o_ref[0, r:r + R2, :] = ((xs - mean) * rstd) * gs + bs


def kernel(x, gamma, beta):
    B, M, L = x.shape
    T = 640
    nj = L // T
    grid = (B, nj)

    return pl.pallas_call(
        _cln_kernel,
        out_shape=jax.ShapeDtypeStruct((B, M, L), x.dtype),
        grid=grid,
        in_specs=[
            pl.BlockSpec((1, M, T), lambda b, j: (b, 0, j)),
            pl.BlockSpec((1, M, 1), lambda b, j: (b, 0, 0)),
            pl.BlockSpec((1, M, 1), lambda b, j: (b, 0, 0)),
        ],
        out_specs=pl.BlockSpec((1, M, T), lambda b, j: (b, 0, j)),
        scratch_shapes=[pltpu.VMEM((2, 1), jnp.float32)],
        compiler_params=pltpu.CompilerParams(
            dimension_semantics=("parallel", "arbitrary"),
        ),
        name="cum_layer_norm",
    )(x, gamma, beta)


# speedup vs baseline: 1.7229x; 1.6875x over previous
import jax
import jax.numpy as jnp
from jax.experimental import pallas as pl
from jax.experimental.pallas import tpu as pltpu

EPS_ = 1e-8


def _cln_kernel(x_ref, g_ref, b_ref, o_ref, carry_ref):
    j = pl.program_id(1)
    _, M, T = x_ref.shape

    R1 = 64
    acc = jnp.zeros((8, T), jnp.float32)
    acc2 = jnp.zeros((8, T), jnp.float32)
    for r in range(0, M, R1):
        xs = x_ref[0, r:r + R1, :].reshape(R1 // 8, 8, T)
        acc = acc + jnp.sum(xs, axis=0)
        acc2 = acc2 + jnp.sum(xs * xs, axis=0)
    s = jnp.sum(acc, axis=0, keepdims=True)
    s2 = jnp.sum(acc2, axis=0, keepdims=True)
    v = jnp.concatenate([s, s2], axis=0)

    k = 1
    while k < T:
        shifted = jnp.concatenate(
            [jnp.zeros((2, k), jnp.float32), v[:, : T - k]], axis=1)
        v = v + shifted
        k *= 2

    @pl.when(j == 0)
    def _():
        carry_ref[...] = jnp.zeros_like(carry_ref)

    v = v + carry_ref[...]
    carry_ref[...] = v[:, T - 1:T]

    pos0 = (j * T + 1).astype(jnp.float32)
    iota = jax.lax.broadcasted_iota(jnp.int32, (1, T), 1).astype(jnp.float32)
    cnt = (iota + pos0) * jnp.float32(M)
    inv = 1.0 / cnt
    mean = v[0:1] * inv
    ex2 = v[1:2] * inv
    var = ex2 - mean * mean
    rstd = jax.lax.rsqrt(var + EPS_)

    R2 = 64
    for r in range(0, M, R2):
        xs = x_ref[0, r:r + R2, :]
        gs = g_ref[0, r:r + R2, :]
        bs = b_ref[0, r:r + R2, :]
        o_ref[0, r:r + R2, :] = ((xs - mean) * rstd) * gs + bs


def kernel(x, gamma, beta):
    B, M, L = x.shape
    T = 3200
    nj = L // T
    grid = (B, nj)

    return pl.pallas_call(
        _cln_kernel,
        out_shape=jax.ShapeDtypeStruct((B, M, L), x.dtype),
        grid=grid,
        in_specs=[
            pl.BlockSpec((1, M, T), lambda b, j: (b, 0, j)),
            pl.BlockSpec((1, M, 1), lambda b, j: (b, 0, 0)),
            pl.BlockSpec((1, M, 1), lambda b, j: (b, 0, 0)),
        ],
        out_specs=pl.BlockSpec((1, M, T), lambda b, j: (b, 0, j)),
        scratch_shapes=[pltpu.VMEM((2, 1), jnp.float32)],
        compiler_params=pltpu.CompilerParams(
            dimension_semantics=("parallel", "arbitrary"),
            vmem_limit_bytes=56 * 1024 * 1024,
        ),
        name="cum_layer_norm",
    )(x, gamma, beta)


# speedup vs baseline: 1.7383x; 1.0089x over previous
import jax
import jax.numpy as jnp
from jax.experimental import pallas as pl
from jax.experimental.pallas import tpu as pltpu

EPS_ = 1e-8


def _cln_kernel(x_ref, g_ref, b_ref, o_ref, carry_ref):
    j = pl.program_id(1)
    _, M, T = x_ref.shape

    R1 = 16
    acc = jnp.zeros((8, T), jnp.float32)
    acc2 = jnp.zeros((8, T), jnp.float32)
    for r in range(0, M, R1):
        xs = x_ref[0, r:r + R1, :].reshape(R1 // 8, 8, T)
        acc = acc + jnp.sum(xs, axis=0)
        acc2 = acc2 + jnp.sum(xs * xs, axis=0)
    s = jnp.sum(acc, axis=0, keepdims=True)
    s2 = jnp.sum(acc2, axis=0, keepdims=True)
    v = jnp.concatenate([s, s2], axis=0)

    k = 1
    while k < T:
        shifted = jnp.concatenate(
            [jnp.zeros((2, k), jnp.float32), v[:, : T - k]], axis=1)
        v = v + shifted
        k *= 2

    @pl.when(j == 0)
    def _():
        carry_ref[...] = jnp.zeros_like(carry_ref)

    v = v + carry_ref[...]
    carry_ref[...] = v[:, T - 1:T]

    pos0 = (j * T + 1).astype(jnp.float32)
    iota = jax.lax.broadcasted_iota(jnp.int32, (1, T), 1).astype(jnp.float32)
    cnt = (iota + pos0) * jnp.float32(M)
    inv = 1.0 / cnt
    mean = v[0:1] * inv
    ex2 = v[1:2] * inv
    var = ex2 - mean * mean
    rstd = jax.lax.rsqrt(var + EPS_)

    R2 = 64
    CW = 640
    for c in range(0, T, CW):
        mc = mean[:, c:c + CW]
        rc = rstd[:, c:c + CW]
        for r in range(0, M, R2):
            xs = x_ref[0, r:r + R2, c:c + CW]
            gs = g_ref[0, r:r + R2, :]
            bs = b_ref[0, r:r + R2, :]
            o_ref[0, r:r + R2, c:c + CW] = ((xs - mc) * rc) * gs + bs


def kernel(x, gamma, beta):
    B, M, L = x.shape
    T = 3200
    nj = L // T
    grid = (B, nj)

    return pl.pallas_call(
        _cln_kernel,
        out_shape=jax.ShapeDtypeStruct((B, M, L), x.dtype),
        grid=grid,
        in_specs=[
            pl.BlockSpec((1, M, T), lambda b, j: (b, 0, j)),
            pl.BlockSpec((1, M, 1), lambda b, j: (b, 0, 0)),
            pl.BlockSpec((1, M, 1), lambda b, j: (b, 0, 0)),
        ],
        out_specs=pl.BlockSpec((1, M, T), lambda b, j: (b, 0, j)),
        scratch_shapes=[pltpu.VMEM((2, 1), jnp.float32)],
        compiler_params=pltpu.CompilerParams(
            dimension_semantics=("parallel", "arbitrary"),
            vmem_limit_bytes=56 * 1024 * 1024,
        ),
        name="cum_layer_norm",
    )(x, gamma, beta)
